# baseline (speedup 1.0000x reference)
import functools
import jax
import jax.numpy as jnp
from jax import lax
from jax.experimental import pallas as pl
from jax.experimental.pallas import tpu as pltpu
from jax.experimental.pallas import tpu_sc as plsc

_N = 5000
_NP = 5120
_CH = 256
_NCH = _NP // _CH
_M = 1024
_TOPN = 1000
_OUTP = 128
_OUT = 100
_SCORE_T = 0.05
_NMS_T = 0.6
_D = 128


def _rank_kernel(srow_ref, scol_ref, idx2_ref, out_ref):
    f32 = jnp.float32
    ir_m = lax.broadcasted_iota(jnp.int32, (1, _M), 1)
    ir_np = lax.broadcasted_iota(jnp.int32, (1, _NP), 1)
    ones_np = jnp.ones((_NP, 1), f32)
    s_row = srow_ref[...]
    cand_row = jnp.where(s_row > _SCORE_T, s_row, 0.0)
    cand_row = jnp.where(ir_np < _N, cand_row, -1.0)
    ir_m_f = ir_m.astype(f32)

    acc = jnp.zeros((_M, 2), f32)
    for k in range(_NCH):
        off = k * _CH
        s_col = scol_ref[off:off + _CH, :]
        idx_col = lax.broadcasted_iota(jnp.int32, (_CH, 1), 0) + off
        cand_col = jnp.where(s_col > _SCORE_T, s_col, 0.0)
        cand_col = jnp.where(idx_col < _N, cand_col, -1.0)
        better = (cand_row > cand_col) | (
            (cand_row == cand_col) & (ir_np < idx_col))
        rank = lax.dot_general(better.astype(f32), ones_np,
                               (((1,), (0,)), ((), ())),
                               preferred_element_type=f32)
        oh = (rank == ir_m_f).astype(f32)
        iv = idx2_ref[off:off + _CH, :]
        acc = acc + lax.dot_general(oh, iv, (((0,), (0,)), ((), ())),
                                    preferred_element_type=f32)
    out_ref[...] = acc


def _sc_gather(vals16, idx):
    info = plsc.get_sparse_core_info()
    nw = info.num_cores * info.num_subcores
    bpw = _M // nw
    mesh = plsc.VectorSubcoreMesh(core_axis_name="c", subcore_axis_name="s")

    @functools.partial(
        pl.kernel, mesh=mesh,
        out_type=jax.ShapeDtypeStruct((_M, _D), jnp.float32),
        scratch_types=[
            pltpu.VMEM((bpw,), jnp.int32),
            pltpu.VMEM((bpw, _D), jnp.float32),
            pltpu.SemaphoreType.DMA,
        ],
    )
    def k(vals_hbm, idx_hbm, out_hbm, idx_v, rows_v, sem):
        wid = lax.axis_index("s") * info.num_cores + lax.axis_index("c")
        base = wid * bpw
        pltpu.sync_copy(idx_hbm.at[pl.ds(base, bpw)], idx_v)
        pltpu.async_copy(vals_hbm.at[idx_v], rows_v, sem).wait()
        pltpu.sync_copy(rows_v, out_hbm.at[pl.ds(base, bpw)])

    return k(vals16, idx)


def _nms_kernel(sorted_ref, ident_ref, out_ref):
    f32 = jnp.float32
    ir_m = lax.broadcasted_iota(jnp.int32, (1, _M), 1)
    ic_m = lax.broadcasted_iota(jnp.int32, (_M, 1), 0)
    ones_m = jnp.ones((_M, 1), f32)
    valid_c = (ic_m < _TOPN).astype(f32)
    valid_r = (ir_m < _TOPN).astype(f32)
    ident = ident_ref[...]
    sv_m = sorted_ref[...] * valid_c
    def trunc16(v):
        bits = jax.lax.bitcast_convert_type(v, jnp.int32)
        return jax.lax.bitcast_convert_type(
            bits & jnp.int32(-65536), f32)

    hi = trunc16(sv_m)
    r = sv_m - hi
    mid = trunc16(r)
    lo = r - mid
    sv48 = jnp.concatenate([hi, mid, lo], axis=1)
    svT48 = lax.dot_general(sv48, ident, (((0,), (0,)), ((), ())),
                            preferred_element_type=f32)
    svT_m = svT48[0:_D, :] + svT48[_D:2 * _D, :] + svT48[2 * _D:, :]

    x1c, y1c = sv_m[:, 0:1], sv_m[:, 1:2]
    x2c, y2c = sv_m[:, 2:3], sv_m[:, 3:4]
    x1r, y1r = svT_m[0:1, :], svT_m[1:2, :]
    x2r, y2r = svT_m[2:3, :], svT_m[3:4, :]
    ix1 = jnp.maximum(x1c, x1r)
    iy1 = jnp.maximum(y1c, y1r)
    ix2 = jnp.minimum(x2c, x2r)
    iy2 = jnp.minimum(y2c, y2r)
    iw = jnp.clip(ix2 - ix1, 0.0)
    ih = jnp.clip(iy2 - iy1, 0.0)
    inter = iw * ih
    area_c = (x2c - x1c) * (y2c - y1c)
    area_r = (x2r - x1r) * (y2r - y1r)
    union = area_c + area_r - inter
    iou = inter / jnp.maximum(union, 1e-9)
    supS = ((iou > _NMS_T) & (ic_m < ir_m)).astype(f32)

    def t_op(x):
        hits = lax.dot_general(supS, x, (((0,), (0,)), ((), ())),
                               preferred_element_type=f32)
        return valid_c * (hits == 0).astype(f32)

    def nms_cond(c):
        x_old, x = c
        return jnp.any(x_old != x)

    def nms_body(c):
        _, x = c
        return x, t_op(x)

    _, keep_col = lax.while_loop(nms_cond, nms_body, (valid_c, t_op(valid_c)))
    keep = lax.dot_general(keep_col, ident, (((0,), (0,)), ((), ())),
                           preferred_element_type=f32)

    ts_col = sv_m[:, 4:5]
    ts_col = ts_col * (ts_col > _SCORE_T).astype(f32)
    ts_row = svT_m[4:5, :]
    ts_row = ts_row * (ts_row > _SCORE_T).astype(f32)
    ks_col = ts_col * keep_col - (1.0 - valid_c)
    ks_row = ts_row * keep - (1.0 - valid_r)
    gtf = (ks_row > ks_col).astype(f32)
    eqf = ((ks_row == ks_col) & (ir_m < ic_m)).astype(f32)
    frank = lax.dot_general(gtf + eqf, ones_m, (((1,), (0,)), ((), ())),
                            preferred_element_type=f32)
    ir_k = lax.broadcasted_iota(jnp.int32, (1, _OUTP), 1).astype(f32)
    ohf = (frank == ir_k).astype(f32)
    col16 = lax.broadcasted_iota(jnp.int32, (1, _D), 1)
    xmat = sv_m * (col16 < 4).astype(f32) \
        + (ts_col * keep_col) * (col16 == 4).astype(f32)
    out_ref[...] = lax.dot_general(ohf, xmat, (((0,), (0,)), ((), ())),
                                   preferred_element_type=f32,
                                   precision=lax.Precision.HIGHEST)


@jax.jit
def kernel(boxes, scores):
    s = scores.astype(jnp.float32)
    b = boxes.astype(jnp.float32)
    s_pad = jnp.full((_NP,), -1.0, jnp.float32).at[:_N].set(s)
    vals16 = jnp.zeros((_NP, _D), jnp.float32)
    vals16 = vals16.at[:_N, :4].set(b).at[:_N, 4].set(s)
    i_np = jnp.arange(_NP, dtype=jnp.int32)
    idx2 = jnp.stack([((i_np // 256) * 256).astype(jnp.float32),
                      (i_np % 256).astype(jnp.float32)], axis=1)
    idx2_s = pl.pallas_call(
        _rank_kernel,
        out_shape=jax.ShapeDtypeStruct((_M, 2), jnp.float32),
    )(s_pad.reshape(1, _NP), s_pad.reshape(_NP, 1), idx2)
    idx = (idx2_s[:, 0] + idx2_s[:, 1]).astype(jnp.int32)
    sorted16 = _sc_gather(vals16, idx)
    iota = jnp.arange(_M, dtype=jnp.int32)
    ident = (iota[:, None] == iota[None, :]).astype(jnp.float32)
    out = pl.pallas_call(
        _nms_kernel,
        out_shape=jax.ShapeDtypeStruct((_OUTP, _D), jnp.float32),
    )(sorted16, ident)
    return out[:_OUT, :5]


# speedup vs baseline: 1.7898x; 1.7898x over previous
import jax
import jax.numpy as jnp
from jax import lax
from jax.experimental import pallas as pl
from jax.experimental.pallas import tpu as pltpu

_N = 5000
_NP = 5120
_CH = 256
_NCH = _NP // _CH
_M = 1024
_TOPN = 1000
_OUTP = 128
_OUT = 100
_SCORE_T = 0.05
_NMS_T = 0.6


def _fcos_kernel(srow_ref, scol_ref, vals_ref, ident_ref, out_ref):
    f32 = jnp.float32
    ir_m = lax.broadcasted_iota(jnp.int32, (1, _M), 1)
    ic_m = lax.broadcasted_iota(jnp.int32, (_M, 1), 0)
    ir_np = lax.broadcasted_iota(jnp.int32, (1, _NP), 1)
    ones_np = jnp.ones((_NP, 1), f32)
    ones_m = jnp.ones((_M, 1), f32)

    s_row = srow_ref[...]
    cand_row = jnp.where(s_row > _SCORE_T, s_row, 0.0)
    cand_row = jnp.where(ir_np < _N, cand_row, -1.0)
    ir_m_f = ir_m.astype(f32)

    ones_ch = jnp.ones((_CH, 1), f32)
    ir_ch = lax.broadcasted_iota(jnp.int32, (1, _CH), 1)
    ic_ch = lax.broadcasted_iota(jnp.int32, (_CH, 1), 0)
    cand_cols = []
    for k in range(_NCH):
        s_col = scol_ref[k * _CH:(k + 1) * _CH, :]
        idx_col = ic_ch + k * _CH
        c = jnp.where(s_col > _SCORE_T, s_col, 0.0)
        cand_cols.append(jnp.where(idx_col < _N, c, -1.0))
    ranks = []
    for a in range(_NCH):
        ca = cand_cols[a]
        row_a = cand_row[:, a * _CH:(a + 1) * _CH]
        diag = (row_a > ca) | ((row_a == ca) & (ir_ch < ic_ch))
        ranks.append(lax.dot_general(diag.astype(f32), ones_ch,
                                     (((1,), (0,)), ((), ())),
                                     preferred_element_type=f32))
    for a in range(_NCH - 1):
        w = _NP - (a + 1) * _CH
        gfull = (cand_row[:, (a + 1) * _CH:] > cand_cols[a]).astype(f32)
        ranks[a] = ranks[a] + lax.dot_general(
            gfull, jnp.ones((w, 1), f32), (((1,), (0,)), ((), ())),
            preferred_element_type=f32)
        colsum = lax.dot_general(gfull, ones_ch, (((0,), (0,)), ((), ())),
                                 preferred_element_type=f32)
        for b in range(a + 1, _NCH):
            off = (b - a - 1) * _CH
            ranks[b] = ranks[b] + (
                float(_CH) - colsum[off:off + _CH, :])

    sv24 = jnp.zeros((_M, 24), f32)
    svT24 = jnp.zeros((24, _M), f32)
    for k in range(_NCH):
        oh = (ranks[k] == ir_m_f).astype(f32)
        v = vals_ref[k * _CH:(k + 1) * _CH, :]
        sv24 = sv24 + lax.dot_general(oh, v, (((0,), (0,)), ((), ())),
                                      preferred_element_type=f32)
        svT24 = svT24 + lax.dot_general(v, oh, (((0,), (0,)), ((), ())),
                                        preferred_element_type=f32)
    sv = sv24[:, 0:8] + sv24[:, 8:16] + sv24[:, 16:24]
    svT = svT24[0:8, :] + svT24[8:16, :] + svT24[16:24, :]

    valid_c = (ic_m < _TOPN).astype(f32)
    valid_r = (ir_m < _TOPN).astype(f32)
    sv_m = sv * valid_c
    svT_m = svT * valid_r

    x1c, y1c = sv_m[:, 0:1], sv_m[:, 1:2]
    x2c, y2c = sv_m[:, 2:3], sv_m[:, 3:4]
    x1r, y1r = svT_m[0:1, :], svT_m[1:2, :]
    x2r, y2r = svT_m[2:3, :], svT_m[3:4, :]
    area_c = (x2c - x1c) * (y2c - y1c)
    area_r = (x2r - x1r) * (y2r - y1r)
    _BQ = 256
    _NQ = _M // _BQ
    lt_q = (lax.broadcasted_iota(jnp.int32, (_BQ, 1), 0)
            < lax.broadcasted_iota(jnp.int32, (1, _BQ), 1))
    row_blocks = []
    for pb in range(_NQ):
        ps = slice(pb * _BQ, (pb + 1) * _BQ)
        blocks = [jnp.zeros((_BQ, _BQ), f32)] * pb
        for qb in range(pb, _NQ):
            qs = slice(qb * _BQ, (qb + 1) * _BQ)
            ix1 = jnp.maximum(x1c[ps], x1r[:, qs])
            iy1 = jnp.maximum(y1c[ps], y1r[:, qs])
            ix2 = jnp.minimum(x2c[ps], x2r[:, qs])
            iy2 = jnp.minimum(y2c[ps], y2r[:, qs])
            inter = jnp.clip(ix2 - ix1, 0.0) * jnp.clip(iy2 - iy1, 0.0)
            union = area_c[ps] + area_r[:, qs] - inter
            iou = inter / jnp.maximum(union, 1e-9)
            hit = iou > _NMS_T
            blocks.append(((hit & lt_q) if qb == pb else hit).astype(f32))
        row_blocks.append(jnp.concatenate(blocks, axis=1))
    supS = jnp.concatenate(row_blocks, axis=0)

    def t_op(x):
        hits = lax.dot_general(supS, x, (((0,), (0,)), ((), ())),
                               preferred_element_type=f32)
        return valid_c * (hits == 0).astype(f32)

    def nms_cond(c):
        x_old, x = c
        return jnp.any(x_old != x)

    def nms_body(c):
        _, x = c
        return x, t_op(x)

    _, keep_col = lax.while_loop(nms_cond, nms_body, (valid_c, t_op(valid_c)))

    keep = lax.dot_general(keep_col, ident_ref[...], (((0,), (0,)), ((), ())),
                           preferred_element_type=f32)
    ts_col = sv_m[:, 4:5]
    ts_col = ts_col * (ts_col > _SCORE_T).astype(f32)
    ts_row = svT_m[4:5, :]
    ts_row = ts_row * (ts_row > _SCORE_T).astype(f32)
    ks_col = ts_col * keep_col - (1.0 - valid_c)
    ks_row = ts_row * keep - (1.0 - valid_r)
    ones_bq = jnp.ones((_BQ, 1), f32)
    gt_pq = (lax.broadcasted_iota(jnp.int32, (_BQ, 1), 0)
             > lax.broadcasted_iota(jnp.int32, (1, _BQ), 1))
    franks = []
    for rb in range(_NQ):
        rs = slice(rb * _BQ, (rb + 1) * _BQ)
        diag = (ks_row[:, rs] > ks_col[rs]) | (
            (ks_row[:, rs] == ks_col[rs]) & gt_pq)
        franks.append(lax.dot_general(diag.astype(f32), ones_bq,
                                      (((1,), (0,)), ((), ())),
                                      preferred_element_type=f32))
    for a in range(_NQ - 1):
        w = _M - (a + 1) * _BQ
        g2 = (ks_row[:, (a + 1) * _BQ:]
              > ks_col[a * _BQ:(a + 1) * _BQ]).astype(f32)
        franks[a] = franks[a] + lax.dot_general(
            g2, jnp.ones((w, 1), f32), (((1,), (0,)), ((), ())),
            preferred_element_type=f32)
        colsum2 = lax.dot_general(g2, ones_bq, (((0,), (0,)), ((), ())),
                                  preferred_element_type=f32)
        for b in range(a + 1, _NQ):
            off = (b - a - 1) * _BQ
            franks[b] = franks[b] + (
                float(_BQ) - colsum2[off:off + _BQ, :])
    frank = jnp.concatenate(franks, axis=0)
    ir_k = lax.broadcasted_iota(jnp.int32, (1, _OUTP), 1).astype(f32)
    ohf = (frank == ir_k).astype(f32)
    col8 = lax.broadcasted_iota(jnp.int32, (1, 8), 1)
    xmat = sv_m * (col8 < 4).astype(f32) \
        + (ts_col * keep_col) * (col8 == 4).astype(f32)
    out_ref[...] = lax.dot_general(ohf, xmat, (((0,), (0,)), ((), ())),
                                   preferred_element_type=f32,
                                   precision=lax.Precision.HIGHEST)


def _postprocess(srow, scol, vals, ident):
    return pl.pallas_call(
        _fcos_kernel,
        out_shape=jax.ShapeDtypeStruct((_OUTP, 8), jnp.float32),
    )(srow, scol, vals, ident)


def _trunc_bf16(v):
    bits = jax.lax.bitcast_convert_type(v, jnp.int32)
    return jax.lax.bitcast_convert_type(
        bits & jnp.int32(-65536), jnp.float32)


def _bf16_split3(v):
    hi = _trunc_bf16(v)
    r = v - hi
    mid = _trunc_bf16(r)
    lo = r - mid
    return hi, mid, lo


@jax.jit
def kernel(boxes, scores):
    s = scores.astype(jnp.float32)
    b = boxes.astype(jnp.float32)
    s_pad = jnp.full((_NP,), -1.0, jnp.float32).at[:_N].set(s)
    vals = jnp.zeros((_NP, 8), jnp.float32)
    vals = vals.at[:_N, :4].set(b).at[:_N, 4].set(s)
    vals24 = jnp.concatenate(_bf16_split3(vals), axis=1)
    iota = jnp.arange(_M, dtype=jnp.int32)
    ident = (iota[:, None] == iota[None, :]).astype(jnp.float32)
    out = _postprocess(s_pad.reshape(1, _NP), s_pad.reshape(_NP, 1),
                       vals24, ident)
    return out[:_OUT, :5]


# speedup vs baseline: 1.9075x; 1.0658x over previous
import jax
import jax.numpy as jnp
from jax import lax
from jax.experimental import pallas as pl
from jax.experimental.pallas import tpu as pltpu

_N = 5000
_NP = 5120
_CH = 256
_NCH = _NP // _CH
_M = 1024
_TOPN = 1000
_OUTP = 128
_OUT = 100
_SCORE_T = 0.05
_NMS_T = 0.6


def _trunc_bf16(v):
    bits = jax.lax.bitcast_convert_type(v, jnp.int32)
    return jax.lax.bitcast_convert_type(
        bits & jnp.int32(-65536), jnp.float32)


def _fcos_kernel(srow_ref, vals_ref, ident_ref, out_ref):
    f32 = jnp.float32
    ir_m = lax.broadcasted_iota(jnp.int32, (1, _M), 1)
    ic_m = lax.broadcasted_iota(jnp.int32, (_M, 1), 0)
    ir_np = lax.broadcasted_iota(jnp.int32, (1, _NP), 1)
    ones_np = jnp.ones((_NP, 1), f32)
    ones_m = jnp.ones((_M, 1), f32)

    s_row = srow_ref[...]
    cand_row = jnp.where(s_row > _SCORE_T, s_row, 0.0)
    cand_row = jnp.where(ir_np < _N, cand_row, -1.0)
    ir_m_f = ir_m.astype(f32)

    ones_ch = jnp.ones((_CH, 1), f32)
    ir_ch = lax.broadcasted_iota(jnp.int32, (1, _CH), 1)
    ic_ch = lax.broadcasted_iota(jnp.int32, (_CH, 1), 0)
    i256 = ident_ref[0:_CH, 0:_CH]
    cand_cols = []
    for k in range(_NCH):
        rchunk = cand_row[:, k * _CH:(k + 1) * _CH]
        hi_c = _trunc_bf16(rchunk)
        r_c = rchunk - hi_c
        mid_c = _trunc_bf16(r_c)
        parts = jnp.concatenate([hi_c, mid_c, r_c - mid_c], axis=0)
        col3 = lax.dot_general(i256, parts.astype(jnp.bfloat16),
                               (((1,), (1,)), ((), ())),
                               preferred_element_type=f32)
        cand_cols.append(col3[:, 0:1] + col3[:, 1:2] + col3[:, 2:3])
    ranks = []
    for a in range(_NCH):
        ca = cand_cols[a]
        row_a = cand_row[:, a * _CH:(a + 1) * _CH]
        diag = (row_a > ca) | ((row_a == ca) & (ir_ch < ic_ch))
        ranks.append(lax.dot_general(diag.astype(f32), ones_ch,
                                     (((1,), (0,)), ((), ())),
                                     preferred_element_type=f32))
    for a in range(_NCH - 1):
        w = _NP - (a + 1) * _CH
        gfull = (cand_row[:, (a + 1) * _CH:] > cand_cols[a]).astype(f32)
        ranks[a] = ranks[a] + lax.dot_general(
            gfull, jnp.ones((w, 1), f32), (((1,), (0,)), ((), ())),
            preferred_element_type=f32)
        colsum = lax.dot_general(gfull, ones_ch, (((0,), (0,)), ((), ())),
                                 preferred_element_type=f32)
        for b in range(a + 1, _NCH):
            off = (b - a - 1) * _CH
            ranks[b] = ranks[b] + (
                float(_CH) - colsum[off:off + _CH, :])

    sv24 = jnp.zeros((_M, 24), f32)
    svT24 = jnp.zeros((24, _M), f32)
    for k in range(_NCH):
        oh = (ranks[k] == ir_m_f).astype(f32)
        v = vals_ref[k * _CH:(k + 1) * _CH, :]
        sv24 = sv24 + lax.dot_general(oh, v, (((0,), (0,)), ((), ())),
                                      preferred_element_type=f32)
        svT24 = svT24 + lax.dot_general(v, oh, (((0,), (0,)), ((), ())),
                                        preferred_element_type=f32)
    sv = sv24[:, 0:8] + sv24[:, 8:16] + sv24[:, 16:24]
    svT = svT24[0:8, :] + svT24[8:16, :] + svT24[16:24, :]

    valid_c = (ic_m < _TOPN).astype(f32)
    valid_r = (ir_m < _TOPN).astype(f32)
    sv_m = sv * valid_c
    svT_m = svT * valid_r

    x1c, y1c = sv_m[:, 0:1], sv_m[:, 1:2]
    x2c, y2c = sv_m[:, 2:3], sv_m[:, 3:4]
    x1r, y1r = svT_m[0:1, :], svT_m[1:2, :]
    x2r, y2r = svT_m[2:3, :], svT_m[3:4, :]
    area_c = (x2c - x1c) * (y2c - y1c)
    area_r = (x2r - x1r) * (y2r - y1r)
    _BQ = 256
    _NQ = _M // _BQ
    lt_q = (lax.broadcasted_iota(jnp.int32, (_BQ, 1), 0)
            < lax.broadcasted_iota(jnp.int32, (1, _BQ), 1))
    row_blocks = []
    for pb in range(_NQ):
        ps = slice(pb * _BQ, (pb + 1) * _BQ)
        blocks = [jnp.zeros((_BQ, _BQ), f32)] * pb
        for qb in range(pb, _NQ):
            qs = slice(qb * _BQ, (qb + 1) * _BQ)
            ix1 = jnp.maximum(x1c[ps], x1r[:, qs])
            iy1 = jnp.maximum(y1c[ps], y1r[:, qs])
            ix2 = jnp.minimum(x2c[ps], x2r[:, qs])
            iy2 = jnp.minimum(y2c[ps], y2r[:, qs])
            inter = jnp.clip(ix2 - ix1, 0.0) * jnp.clip(iy2 - iy1, 0.0)
            union = area_c[ps] + area_r[:, qs] - inter
            iou = inter / jnp.maximum(union, 1e-9)
            hit = iou > _NMS_T
            blocks.append(((hit & lt_q) if qb == pb else hit).astype(f32))
        row_blocks.append(jnp.concatenate(blocks, axis=1))
    supS = jnp.concatenate(row_blocks, axis=0)

    def t_op(x):
        hits = lax.dot_general(supS, x, (((0,), (0,)), ((), ())),
                               preferred_element_type=f32)
        return valid_c * (hits == 0).astype(f32)

    def nms_cond(c):
        x_old, x = c
        return jnp.any(x_old != x)

    def nms_body(c):
        _, x = c
        return x, t_op(x)

    _, keep_col = lax.while_loop(nms_cond, nms_body, (valid_c, t_op(valid_c)))

    keep = lax.dot_general(keep_col.astype(jnp.bfloat16), ident_ref[...],
                           (((0,), (0,)), ((), ())),
                           preferred_element_type=f32)
    ts_col = sv_m[:, 4:5]
    ts_col = ts_col * (ts_col > _SCORE_T).astype(f32)
    ts_row = svT_m[4:5, :]
    ts_row = ts_row * (ts_row > _SCORE_T).astype(f32)
    ks_col = ts_col * keep_col - (1.0 - valid_c)
    ks_row = ts_row * keep - (1.0 - valid_r)
    ones_bq = jnp.ones((_BQ, 1), f32)
    gt_pq = (lax.broadcasted_iota(jnp.int32, (_BQ, 1), 0)
             > lax.broadcasted_iota(jnp.int32, (1, _BQ), 1))
    franks = []
    for rb in range(_NQ):
        rs = slice(rb * _BQ, (rb + 1) * _BQ)
        diag = (ks_row[:, rs] > ks_col[rs]) | (
            (ks_row[:, rs] == ks_col[rs]) & gt_pq)
        franks.append(lax.dot_general(diag.astype(f32), ones_bq,
                                      (((1,), (0,)), ((), ())),
                                      preferred_element_type=f32))
    for a in range(_NQ - 1):
        w = _M - (a + 1) * _BQ
        g2 = (ks_row[:, (a + 1) * _BQ:]
              > ks_col[a * _BQ:(a + 1) * _BQ]).astype(f32)
        franks[a] = franks[a] + lax.dot_general(
            g2, jnp.ones((w, 1), f32), (((1,), (0,)), ((), ())),
            preferred_element_type=f32)
        colsum2 = lax.dot_general(g2, ones_bq, (((0,), (0,)), ((), ())),
                                  preferred_element_type=f32)
        for b in range(a + 1, _NQ):
            off = (b - a - 1) * _BQ
            franks[b] = franks[b] + (
                float(_BQ) - colsum2[off:off + _BQ, :])
    frank = jnp.concatenate(franks, axis=0)
    ir_k = lax.broadcasted_iota(jnp.int32, (1, _OUTP), 1).astype(f32)
    ohf = (frank == ir_k).astype(f32)
    col8 = lax.broadcasted_iota(jnp.int32, (1, 8), 1)
    xmat = sv_m * (col8 < 4).astype(f32) \
        + (ts_col * keep_col) * (col8 == 4).astype(f32)
    out_ref[...] = lax.dot_general(ohf, xmat, (((0,), (0,)), ((), ())),
                                   preferred_element_type=f32,
                                   precision=lax.Precision.HIGHEST)


def _postprocess(srow, vals, ident):
    return pl.pallas_call(
        _fcos_kernel,
        out_shape=jax.ShapeDtypeStruct((_OUTP, 8), jnp.float32),
    )(srow, vals, ident)


def _bf16_split3(v):
    hi = _trunc_bf16(v)
    r = v - hi
    mid = _trunc_bf16(r)
    lo = r - mid
    return hi, mid, lo


@jax.jit
def kernel(boxes, scores):
    s = scores.astype(jnp.float32)
    b = boxes.astype(jnp.float32)
    s_pad = jnp.full((_NP,), -1.0, jnp.float32).at[:_N].set(s)
    vals = jnp.zeros((_NP, 8), jnp.float32)
    vals = vals.at[:_N, :4].set(b).at[:_N, 4].set(s)
    vals24 = jnp.concatenate(_bf16_split3(vals), axis=1)
    iota = jnp.arange(_M, dtype=jnp.int32)
    ident = (iota[:, None] == iota[None, :]).astype(jnp.bfloat16)
    out = _postprocess(s_pad.reshape(1, _NP), vals24, ident)
    return out[:_OUT, :5]


# speedup vs baseline: 1.9720x; 1.0338x over previous
import jax
import jax.numpy as jnp
from jax import lax
from jax.experimental import pallas as pl
from jax.experimental.pallas import tpu as pltpu

_N = 5000
_NP = 5120
_CH = 256
_NCH = _NP // _CH
_M = 1024
_TOPN = 1000
_OUTP = 128
_OUT = 100
_SCORE_T = 0.05
_NMS_T = 0.6


def _trunc_bf16(v):
    bits = jax.lax.bitcast_convert_type(v, jnp.int32)
    return jax.lax.bitcast_convert_type(
        bits & jnp.int32(-65536), jnp.float32)


def _fcos_kernel(srow_ref, vals_ref, ident_ref, out_ref):
    f32 = jnp.float32
    ir_m = lax.broadcasted_iota(jnp.int32, (1, _M), 1)
    ic_m = lax.broadcasted_iota(jnp.int32, (_M, 1), 0)
    ir_np = lax.broadcasted_iota(jnp.int32, (1, _NP), 1)
    ones_np = jnp.ones((_NP, 1), f32)
    ones_m = jnp.ones((_M, 1), f32)

    s_row = srow_ref[...]
    cand_row = jnp.where(s_row > _SCORE_T, s_row, 0.0)
    cand_row = jnp.where(ir_np < _N, cand_row, -1.0)
    ir_m_f = ir_m.astype(f32)

    ones_ch = jnp.ones((_CH, 1), f32)
    ir_ch = lax.broadcasted_iota(jnp.int32, (1, _CH), 1)
    ic_ch = lax.broadcasted_iota(jnp.int32, (_CH, 1), 0)
    i256 = ident_ref[...]
    cand_cols = []
    for k in range(_NCH):
        rchunk = cand_row[:, k * _CH:(k + 1) * _CH]
        hi_c = _trunc_bf16(rchunk)
        r_c = rchunk - hi_c
        mid_c = _trunc_bf16(r_c)
        parts = jnp.concatenate([hi_c, mid_c, r_c - mid_c], axis=0)
        col3 = lax.dot_general(i256, parts.astype(jnp.bfloat16),
                               (((1,), (1,)), ((), ())),
                               preferred_element_type=f32)
        cand_cols.append(col3[:, 0:1] + col3[:, 1:2] + col3[:, 2:3])
    ranks = []
    for a in range(_NCH):
        ca = cand_cols[a]
        row_a = cand_row[:, a * _CH:(a + 1) * _CH]
        diag = (row_a > ca) | ((row_a == ca) & (ir_ch < ic_ch))
        ranks.append(lax.dot_general(diag.astype(f32), ones_ch,
                                     (((1,), (0,)), ((), ())),
                                     preferred_element_type=f32))
    for a in range(_NCH - 1):
        w = _NP - (a + 1) * _CH
        gfull = (cand_row[:, (a + 1) * _CH:] > cand_cols[a]).astype(f32)
        ranks[a] = ranks[a] + lax.dot_general(
            gfull, jnp.ones((w, 1), f32), (((1,), (0,)), ((), ())),
            preferred_element_type=f32)
        colsum = lax.dot_general(gfull, ones_ch, (((0,), (0,)), ((), ())),
                                 preferred_element_type=f32)
        for b in range(a + 1, _NCH):
            off = (b - a - 1) * _CH
            ranks[b] = ranks[b] + (
                float(_CH) - colsum[off:off + _CH, :])

    sv24 = jnp.zeros((_M, 24), f32)
    svT24 = jnp.zeros((24, _M), f32)
    for k in range(_NCH):
        oh = (ranks[k] == ir_m_f).astype(f32)
        v = vals_ref[k * _CH:(k + 1) * _CH, :]
        sv24 = sv24 + lax.dot_general(oh, v, (((0,), (0,)), ((), ())),
                                      preferred_element_type=f32)
        svT24 = svT24 + lax.dot_general(v, oh, (((0,), (0,)), ((), ())),
                                        preferred_element_type=f32)
    sv = sv24[:, 0:8] + sv24[:, 8:16] + sv24[:, 16:24]
    svT = svT24[0:8, :] + svT24[8:16, :] + svT24[16:24, :]

    valid_c = (ic_m < _TOPN).astype(f32)
    valid_r = (ir_m < _TOPN).astype(f32)
    sv_m = sv * valid_c
    svT_m = svT * valid_r

    x1c, y1c = sv_m[:, 0:1], sv_m[:, 1:2]
    x2c, y2c = sv_m[:, 2:3], sv_m[:, 3:4]
    x1r, y1r = svT_m[0:1, :], svT_m[1:2, :]
    x2r, y2r = svT_m[2:3, :], svT_m[3:4, :]
    area_c = (x2c - x1c) * (y2c - y1c)
    area_r = (x2r - x1r) * (y2r - y1r)
    _BQ = 256
    _NQ = _M // _BQ
    lt_q = (lax.broadcasted_iota(jnp.int32, (_BQ, 1), 0)
            < lax.broadcasted_iota(jnp.int32, (1, _BQ), 1))
    row_blocks = []
    for pb in range(_NQ):
        ps = slice(pb * _BQ, (pb + 1) * _BQ)
        blocks = [jnp.zeros((_BQ, _BQ), f32)] * pb
        for qb in range(pb, _NQ):
            qs = slice(qb * _BQ, (qb + 1) * _BQ)
            ix1 = jnp.maximum(x1c[ps], x1r[:, qs])
            iy1 = jnp.maximum(y1c[ps], y1r[:, qs])
            ix2 = jnp.minimum(x2c[ps], x2r[:, qs])
            iy2 = jnp.minimum(y2c[ps], y2r[:, qs])
            inter = jnp.clip(ix2 - ix1, 0.0) * jnp.clip(iy2 - iy1, 0.0)
            union = area_c[ps] + area_r[:, qs] - inter
            iou = inter / jnp.maximum(union, 1e-9)
            hit = iou > _NMS_T
            blocks.append(((hit & lt_q) if qb == pb else hit).astype(f32))
        row_blocks.append(jnp.concatenate(blocks, axis=1))
    supS = jnp.concatenate(row_blocks, axis=0)

    def t_op(x):
        hits = lax.dot_general(supS, x, (((0,), (0,)), ((), ())),
                               preferred_element_type=f32)
        return valid_c * (hits == 0).astype(f32)

    def nms_cond(c):
        x_old, x = c
        return jnp.any(x_old != x)

    def nms_body(c):
        _, x = c
        return x, t_op(x)

    _, keep_col = lax.while_loop(nms_cond, nms_body, (valid_c, t_op(valid_c)))

    keep = jnp.concatenate(
        [lax.dot_general(keep_col[q * _CH:(q + 1) * _CH, :].astype(
            jnp.bfloat16), i256, (((0,), (0,)), ((), ())),
            preferred_element_type=f32)
         for q in range(_M // _CH)], axis=1)
    ts_col = sv_m[:, 4:5]
    ts_col = ts_col * (ts_col > _SCORE_T).astype(f32)
    ts_row = svT_m[4:5, :]
    ts_row = ts_row * (ts_row > _SCORE_T).astype(f32)
    ks_col = ts_col * keep_col - (1.0 - valid_c)
    ks_row = ts_row * keep - (1.0 - valid_r)
    ones_bq = jnp.ones((_BQ, 1), f32)
    gt_pq = (lax.broadcasted_iota(jnp.int32, (_BQ, 1), 0)
             > lax.broadcasted_iota(jnp.int32, (1, _BQ), 1))
    franks = []
    for rb in range(_NQ):
        rs = slice(rb * _BQ, (rb + 1) * _BQ)
        diag = (ks_row[:, rs] > ks_col[rs]) | (
            (ks_row[:, rs] == ks_col[rs]) & gt_pq)
        franks.append(lax.dot_general(diag.astype(f32), ones_bq,
                                      (((1,), (0,)), ((), ())),
                                      preferred_element_type=f32))
    for a in range(_NQ - 1):
        w = _M - (a + 1) * _BQ
        g2 = (ks_row[:, (a + 1) * _BQ:]
              > ks_col[a * _BQ:(a + 1) * _BQ]).astype(f32)
        franks[a] = franks[a] + lax.dot_general(
            g2, jnp.ones((w, 1), f32), (((1,), (0,)), ((), ())),
            preferred_element_type=f32)
        colsum2 = lax.dot_general(g2, ones_bq, (((0,), (0,)), ((), ())),
                                  preferred_element_type=f32)
        for b in range(a + 1, _NQ):
            off = (b - a - 1) * _BQ
            franks[b] = franks[b] + (
                float(_BQ) - colsum2[off:off + _BQ, :])
    frank = jnp.concatenate(franks, axis=0)
    ir_k = lax.broadcasted_iota(jnp.int32, (1, _OUTP), 1).astype(f32)
    ohf = (frank == ir_k).astype(f32)
    col8 = lax.broadcasted_iota(jnp.int32, (1, 8), 1)
    xmat = sv_m * (col8 < 4).astype(f32) \
        + (ts_col * keep_col) * (col8 == 4).astype(f32)
    out_ref[...] = lax.dot_general(ohf, xmat, (((0,), (0,)), ((), ())),
                                   preferred_element_type=f32,
                                   precision=lax.Precision.HIGHEST)


def _postprocess(srow, vals, ident):
    return pl.pallas_call(
        _fcos_kernel,
        out_shape=jax.ShapeDtypeStruct((_OUTP, 8), jnp.float32),
    )(srow, vals, ident)


def _bf16_split3(v):
    hi = _trunc_bf16(v)
    r = v - hi
    mid = _trunc_bf16(r)
    lo = r - mid
    return hi, mid, lo


@jax.jit
def kernel(boxes, scores):
    s = scores.astype(jnp.float32)
    b = boxes.astype(jnp.float32)
    s_pad = jnp.full((_NP,), -1.0, jnp.float32).at[:_N].set(s)
    vals = jnp.zeros((_NP, 8), jnp.float32)
    vals = vals.at[:_N, :4].set(b).at[:_N, 4].set(s)
    vals24 = jnp.concatenate(_bf16_split3(vals), axis=1)
    iota = jnp.arange(_CH, dtype=jnp.int32)
    ident = (iota[:, None] == iota[None, :]).astype(jnp.bfloat16)
    out = _postprocess(s_pad.reshape(1, _NP), vals24, ident)
    return out[:_OUT, :5]
